# baseline (speedup 1.0000x reference)
import functools

import jax
import jax.numpy as jnp
from jax import lax
from jax.experimental import pallas as pl
from jax.experimental.pallas import tpu as pltpu
from jax.experimental.pallas import tpu_sc as plsc

BATCH = 16
SEQ_LEN = 2048
NUM_LAYERS = 3
DIM = 64
NUM_POS = BATCH * SEQ_LEN
VOCAB = 100000

_info = plsc.get_sparse_core_info()
_NC, _NS = _info.num_cores, _info.num_subcores
NW = _NC * _NS
POS_PER_W = NUM_POS // NW
CHUNK = 128
NCHUNK = POS_PER_W // CHUNK

_mesh = plsc.VectorSubcoreMesh(core_axis_name="c", subcore_axis_name="s")


@functools.partial(
    pl.kernel,
    mesh=_mesh,
    out_type=jax.ShapeDtypeStruct((NUM_POS, NUM_LAYERS * DIM), jnp.float32),
    scratch_types=[
        pltpu.VMEM((CHUNK,), jnp.int32),
        pltpu.VMEM((CHUNK,), jnp.int32),
        pltpu.VMEM((CHUNK, 2 * DIM), jnp.float32),
        pltpu.VMEM((CHUNK, NUM_LAYERS * DIM), jnp.float32),
        pltpu.SemaphoreType.DMA,
    ],
)
def _emb_lookup(tok_hbm, w6, w9, w12, out_hbm, tok_v, ixp_v, g_v, stage_v, sem):
    wid = lax.axis_index("c") * _NS + lax.axis_index("s")
    base = wid * POS_PER_W
    tables = (w6, w9, w12)

    def chunk_body(c, carry):
        p0 = base + c * CHUNK
        for i in range(NUM_LAYERS):
            pltpu.sync_copy(tok_hbm.at[pl.ds(i * NUM_POS + p0, CHUNK)], tok_v)

            def shift(j, cc):
                t = tok_v[pl.ds(j * 16, 16)]
                ixp_v[pl.ds(j * 16, 16)] = t >> 1
                return cc

            lax.fori_loop(0, CHUNK // 16, shift, 0)
            pltpu.async_copy(tables[i].at[ixp_v], g_v, sem).wait()

            def select(j, cc, i=i):
                tv = tok_v[pl.ds(j * 16, 16)]
                pv = (tv & 1) * DIM
                for lane in range(16):
                    off = pv[lane]
                    p = j * 16 + lane
                    for kk in range(DIM // 16):
                        stage_v[p, pl.ds(i * DIM + kk * 16, 16)] = g_v[
                            p, pl.ds(off + kk * 16, 16)
                        ]
                return cc

            lax.fori_loop(0, CHUNK // 16, select, 0)
        pltpu.sync_copy(stage_v, out_hbm.at[pl.ds(p0, CHUNK)])
        return carry

    lax.fori_loop(0, NCHUNK, chunk_body, 0)


def kernel(tokens, W6, W9, W12):
    tok_t = jnp.transpose(
        tokens.reshape(NUM_POS, NUM_LAYERS).astype(jnp.int32)
    ).reshape(NUM_LAYERS * NUM_POS)
    out = _emb_lookup(
        tok_t,
        W6.reshape(VOCAB // 2, 2 * DIM),
        W9.reshape(VOCAB // 2, 2 * DIM),
        W12.reshape(VOCAB // 2, 2 * DIM),
    )
    return out.reshape(BATCH, SEQ_LEN, NUM_LAYERS, DIM)
